# baseline (speedup 1.0000x reference)
import functools

import jax
import jax.numpy as jnp
from jax import lax
from jax.experimental import pallas as pl
from jax.experimental.pallas import tpu as pltpu
from jax.experimental.pallas import tpu_sc as plsc

F32 = jnp.float32
I32 = jnp.int32

NC = 2
NS = 16
NW = NC * NS
CH = 128
CW = 8
NBUF = 2
BN = 1000


def _sc_mesh():
  return plsc.VectorSubcoreMesh(core_axis_name="c", subcore_axis_name="s")


@functools.lru_cache(None)
def _deg_call(nacc, nch):
  spl = nacc // NS

  def body(srcd, dstd, ones_h, zeros_h, out, acc_o, acc_i):
    cid = lax.axis_index("c")
    sid = lax.axis_index("s")
    wid = sid * NC + cid

    def inner(idxs, idxd, ones_v, zeros_v):
      pltpu.sync_copy(ones_h, ones_v)
      pltpu.sync_copy(zeros_h, zeros_v)
      base = sid * spl
      pltpu.sync_copy(zeros_v, acc_o.at[pl.ds(base, spl)])
      pltpu.sync_copy(zeros_v, acc_i.at[pl.ds(base, spl)])
      plsc.subcore_barrier()
      pltpu.sync_copy(srcd.at[wid], idxs)
      pltpu.sync_copy(dstd.at[wid], idxd)

      def step(j, carry):
        pltpu.sync_copy(ones_v, acc_o.at[idxs.at[j]], add=True)
        pltpu.sync_copy(ones_v, acc_i.at[idxd.at[j]], add=True)
        return carry

      lax.fori_loop(0, nch, step, 0)
      plsc.subcore_barrier()
      pltpu.sync_copy(acc_o.at[pl.ds(base, spl)],
                      out.at[pl.ds((cid * 2) * nacc + base, spl)])
      pltpu.sync_copy(acc_i.at[pl.ds(base, spl)],
                      out.at[pl.ds((cid * 2 + 1) * nacc + base, spl)])

    pl.run_scoped(inner,
                  pltpu.VMEM((nch, CH), I32),
                  pltpu.VMEM((nch, CH), I32),
                  pltpu.VMEM((CH,), F32),
                  pltpu.VMEM((spl,), F32))

  return pl.kernel(
      body,
      out_type=jax.ShapeDtypeStruct((4 * nacc,), F32),
      mesh=_sc_mesh(),
      scratch_types=[
          pltpu.VMEM_SHARED((nacc,), F32),
          pltpu.VMEM_SHARED((nacc,), F32),
      ],
  )


@functools.lru_cache(None)
def _conv_call(n, d, nacc, nch):
  spl = nacc // NS
  nz = spl // CH

  def body(xs, srcg, dstd, zeros_h, out, acc):
    cid = lax.axis_index("c")
    sid = lax.axis_index("s")
    wid = sid * NC + cid

    nwin = nch // CW

    def inner(idxs, idxd, rows, semg, sems0, sems1):
      sems = [sems0, sems1]
      rbase = sid * spl
      for k in range(nz):
        pltpu.sync_copy(zeros_h, acc.at[pl.ds(rbase + k * CH, CH)])
      plsc.subcore_barrier()

      def gstart(wp, k, b):
        pltpu.async_copy(xs.at[idxs.at[wp, k]], rows.at[b], semg)

      def gwait(b):
        pltpu.make_async_copy(xs.at[idxs.at[0, 0]], rows.at[b], semg).wait()

      def sstart(wp, k, b):
        pltpu.async_copy(rows.at[b], acc.at[idxd.at[wp, k]], sems[b], add=True)

      def swait(b):
        pltpu.make_async_copy(rows.at[b], acc.at[idxd.at[0, 0]],
                              sems[b]).wait()

      pltpu.sync_copy(srcg.at[wid, pl.ds(0, CW)], idxs.at[0])
      pltpu.sync_copy(dstd.at[wid, pl.ds(0, CW)], idxd.at[0])
      gstart(0, 0, 0)

      def win(w, carry):
        wp = w % 2

        @pl.when(w + 1 < nwin)
        def _():
          pltpu.sync_copy(srcg.at[wid, pl.ds((w + 1) * CW, CW)],
                          idxs.at[1 - wp])
          pltpu.sync_copy(dstd.at[wid, pl.ds((w + 1) * CW, CW)],
                          idxd.at[1 - wp])

        for k in range(CW):
          b = k % NBUF
          gwait(b)
          if k == 0:
            @pl.when(w > 0)
            def _():
              swait(1 - b)
          else:
            swait(1 - b)
          if k + 1 < CW:
            gstart(wp, k + 1, 1 - b)
          else:
            @pl.when(w + 1 < nwin)
            def _():
              gstart(1 - wp, 0, 1 - b)
          sstart(wp, k, b)
        return carry

      lax.fori_loop(0, nwin, win, 0)
      swait((CW - 1) % NBUF)
      plsc.subcore_barrier()
      obase = cid * nacc + rbase
      for k in range(nz):
        pltpu.sync_copy(acc.at[pl.ds(rbase + k * CH, CH)],
                        out.at[pl.ds(obase + k * CH, CH)])

    pl.run_scoped(inner,
                  pltpu.VMEM((2, CW, CH), I32),
                  pltpu.VMEM((2, CW, CH), I32),
                  pltpu.VMEM((NBUF, CH, d), F32),
                  pltpu.SemaphoreType.DMA,
                  pltpu.SemaphoreType.DMA,
                  pltpu.SemaphoreType.DMA)

  return pl.kernel(
      body,
      out_type=jax.ShapeDtypeStruct((2 * nacc, d), F32),
      mesh=_sc_mesh(),
      compiler_params=pltpu.CompilerParams(use_tc_tiling_on_sc=True),
      scratch_types=[
          pltpu.VMEM_SHARED((nacc, d), F32),
      ],
  )


_BN_SCALE = float(1.0 / (1.0 + 1e-5) ** 0.5)
_ATT_SCALE = float(1.0 / 1280.0 ** 0.5)


def _tc1_body(h_ref, deg_ref, g_ref, b_ref, gid_ref, xs_ref, ia_ref,
              iacc, cnt):
  i = pl.program_id(0)
  nb = pl.num_programs(0)
  dsl = deg_ref[0]
  inv_out = lax.rsqrt(jnp.maximum(dsl[0] + dsl[2], 1.0))
  hb = h_ref[...]
  bnh = hb * (g_ref[0] * _BN_SCALE) + b_ref[0]
  xs_ref[...] = bnh * inv_out[:, None]

  gid = gid_ref[0, 0]
  ohf = (gid[:, None] == lax.broadcasted_iota(I32, (BN, 128), 1)).astype(F32)

  @pl.when(i == 0)
  def _():
    iacc[...] = jnp.zeros((128, 128), F32)
    cnt[...] = jnp.zeros((8, 128), F32)

  iacc[...] = iacc[...] + lax.dot_general(
      ohf, hb, (((0,), (0,)), ((), ())), preferred_element_type=F32)
  cnt[0:1, :] = cnt[0:1, :] + jnp.sum(ohf, axis=0, keepdims=True)

  @pl.when(i == nb - 1)
  def _():
    ia_ref[...] = iacc[...] / jnp.maximum(cnt[0], 1.0)[:, None]


def _tc2_body(h_ref, p_ref, deg_ref, w1_ref, b1_ref, g2_ref, b2_ref,
              hh_ref, xs2_ref):
  dsl = deg_ref[0]
  inv_out = lax.rsqrt(jnp.maximum(dsl[0] + dsl[2], 1.0))
  inv_in = lax.rsqrt(jnp.maximum(dsl[1] + dsl[3], 1.0))
  p = p_ref[...]
  agg = (p[0] + p[1]) * inv_in[:, None]
  z = jnp.dot(agg, w1_ref[...], preferred_element_type=F32) + b1_ref[0]
  hh = h_ref[...] + jnp.maximum(z, 0.0)
  hh_ref[...] = hh
  xs2_ref[...] = (hh * (g2_ref[0] * _BN_SCALE) + b2_ref[0]) * inv_out[:, None]


def _tc3_body(p_ref, hh_ref, deg_ref, w2_ref, b2_ref, wq_ref, wk_ref, wv_ref,
              if_ref, gid_ref, hh2_ref, v_ref, att_ref, mp_ref):
  dsl = deg_ref[0]
  inv_in = lax.rsqrt(jnp.maximum(dsl[1] + dsl[3], 1.0))
  p = p_ref[...]
  agg = (p[0] + p[1]) * inv_in[:, None]
  z = jnp.dot(agg, w2_ref[...], preferred_element_type=F32) + b2_ref[0]
  hh2 = hh_ref[...] + jnp.maximum(z, 0.0)
  q = jnp.dot(hh2, wq_ref[...], preferred_element_type=F32)
  v = jnp.dot(hh2, wv_ref[...], preferred_element_type=F32)
  kg = jnp.dot(if_ref[...], wk_ref[...], preferred_element_type=F32)
  gid = gid_ref[0, 0]
  oh = gid[:, None] == lax.broadcasted_iota(I32, (BN, 128), 1)
  ohf = oh.astype(F32)
  kn = jnp.dot(ohf, kg, preferred_element_type=F32)
  att = jnp.sum(q * kn, axis=1) * _ATT_SCALE
  att_ref[0, 0] = att
  mp_ref[0, 0] = jnp.max(jnp.where(oh, att[:, None], -1e30), axis=0)
  hh2_ref[...] = hh2
  v_ref[...] = v


def _ln(x, g, b):
  mu = jnp.mean(x, axis=-1, keepdims=True)
  dxm = x - mu
  var = jnp.mean(dxm * dxm, axis=-1, keepdims=True)
  return dxm * lax.rsqrt(var + 1e-5) * g + b


def _tc4a_body(att_ref, gid_ref, mp_ref, ms_ref, stat):
  i = pl.program_id(0)
  nbs = pl.num_programs(0)
  bn2 = att_ref.shape[2]
  gid = gid_ref[0, 0]
  ohf = (gid[:, None] == lax.broadcasted_iota(I32, (bn2, 128), 1)).astype(F32)

  @pl.when(i == 0)
  def _():
    stat[0:1, :] = jnp.max(mp_ref[:, 0, :], axis=0, keepdims=True)
    stat[1:2, :] = jnp.zeros((1, 128), F32)

  att = att_ref[0, 0]
  mg = jnp.dot(ohf, stat[0], preferred_element_type=F32)
  e = jnp.exp(att - mg)
  stat[1:2, :] = stat[1:2, :] + jnp.dot(e, ohf,
                                        preferred_element_type=F32)[None, :]

  @pl.when(i == nbs - 1)
  def _():
    ms_ref[...] = stat[0:2, :]


def _tc4b_body(hh2_ref, v_ref, att_ref, gid_ref, ms_ref, wc_ref,
               fw1_ref, fb1_ref, fw2_ref, fb2_ref, lg_ref, lb_ref,
               ro_ref, racc):
  i = pl.program_id(0)
  nb = pl.num_programs(0)
  gid = gid_ref[0, 0]
  ohf = (gid[:, None] == lax.broadcasted_iota(I32, (BN, 128), 1)).astype(F32)

  @pl.when(i == 0)
  def _():
    racc[...] = jnp.zeros((128, 128), F32)

  att = att_ref[0, 0]
  mg = jnp.dot(ohf, ms_ref[0], preferred_element_type=F32)
  e = jnp.exp(att - mg)
  s_g = jnp.dot(ohf, ms_ref[1], preferred_element_type=F32)
  alpha = e / s_g
  tp = v_ref[...] * alpha[:, None]
  mo = jnp.dot(tp, wc_ref[...], preferred_element_type=F32)
  mo = _ln(mo + hh2_ref[...], lg_ref[0], lb_ref[0])
  ffo = jnp.dot(
      jnp.maximum(
          jnp.dot(mo, fw1_ref[...], preferred_element_type=F32) + fb1_ref[0],
          0.0),
      fw2_ref[...], preferred_element_type=F32) + fb2_ref[0]
  mo2 = _ln(ffo + mo, lg_ref[0], lb_ref[0])
  racc[...] = racc[...] + lax.dot_general(
      ohf, mo2, (((0,), (0,)), ((), ())), preferred_element_type=F32)

  @pl.when(i == nb - 1)
  def _():
    ro_ref[...] = racc[...]


def _row_spec(d):
  return pl.BlockSpec((BN, d), lambda i: (i, 0))


def _full_spec(shape):
  return pl.BlockSpec(shape, lambda i: tuple(0 for _ in shape))


def kernel(h, inter_f, edge_index, graph_ids, W1, b1, W2, b2, bn1_g, bn1_b,
           bn2_g, bn2_b, Wq, Wk, Wv, Wc, ffW1, ffb1, ffW2, ffb2, ln_g, ln_b):
  n, d = h.shape
  g = inter_f.shape[0]
  e = edge_index.shape[1]
  nb = n // BN

  nacc = -(-(n + 64) // (NS * CH)) * (NS * CH)
  nch = -(-e // (NW * CH))
  nch = -(-nch // CW) * CW
  ep = NW * nch * CH
  pad = ep - e

  src = edge_index[0].astype(I32)
  dst = edge_index[1].astype(I32)
  padi = jnp.arange(pad, dtype=I32) % 32
  srcg = jnp.concatenate([src, padi]).reshape(NW, nch, CH)
  srcd = jnp.concatenate([src, n + padi]).reshape(NW, nch, CH)
  dstd = jnp.concatenate([dst, n + 32 + padi]).reshape(NW, nch, CH)

  deg = _deg_call(nacc, nch)(
      srcd, dstd, jnp.ones((CH,), F32), jnp.zeros((nacc // NS,), F32))
  deg = deg.reshape(4, nacc)[:, :n].reshape(4, nb, BN).transpose(1, 0, 2)

  deg_spec = pl.BlockSpec((1, 4, BN), lambda i: (i, 0, 0))
  vec_spec = _full_spec((1, d))
  gid_r = graph_ids.astype(I32).reshape(nb, 1, BN)
  blk_spec = pl.BlockSpec((1, 1, BN), lambda i: (i, 0, 0))

  xs1, ia = pl.pallas_call(
      _tc1_body,
      grid=(nb,),
      in_specs=[_row_spec(d), deg_spec, vec_spec, vec_spec, blk_spec],
      out_specs=[_row_spec(d), _full_spec((128, 128))],
      out_shape=[jax.ShapeDtypeStruct((n, d), F32),
                 jax.ShapeDtypeStruct((128, 128), F32)],
      scratch_shapes=[pltpu.VMEM((128, 128), F32), pltpu.VMEM((8, 128), F32)],
  )(h, deg, bn1_g.reshape(1, d), bn1_b.reshape(1, d), gid_r)

  zeros_chd = jnp.zeros((CH, d), F32)
  p1 = _conv_call(n, d, nacc, nch)(xs1, srcg, dstd, zeros_chd)
  p1 = p1.reshape(2, nacc, d)
  p_spec = pl.BlockSpec((2, BN, d), lambda i: (0, i, 0))

  hh, xs2 = pl.pallas_call(
      _tc2_body,
      grid=(nb,),
      in_specs=[_row_spec(d), p_spec, deg_spec, _full_spec((d, d)), vec_spec,
                vec_spec, vec_spec],
      out_specs=[_row_spec(d), _row_spec(d)],
      out_shape=[jax.ShapeDtypeStruct((n, d), F32),
                 jax.ShapeDtypeStruct((n, d), F32)],
  )(h, p1, deg, W1, b1.reshape(1, d), bn2_g.reshape(1, d),
    bn2_b.reshape(1, d))

  p2 = _conv_call(n, d, nacc, nch)(xs2, srcg, dstd, zeros_chd)
  p2 = p2.reshape(2, nacc, d)

  if_p = jnp.zeros((128, d), F32).at[:g].set(inter_f)

  hh2, v, att, mp = pl.pallas_call(
      _tc3_body,
      grid=(nb,),
      in_specs=[p_spec, _row_spec(d), deg_spec, _full_spec((d, d)), vec_spec,
                _full_spec((d, d)), _full_spec((d, d)), _full_spec((d, d)),
                _full_spec((128, d)), blk_spec],
      out_specs=[_row_spec(d), _row_spec(d), blk_spec,
                 pl.BlockSpec((1, 1, 128), lambda i: (i, 0, 0))],
      out_shape=[jax.ShapeDtypeStruct((n, d), F32),
                 jax.ShapeDtypeStruct((n, d), F32),
                 jax.ShapeDtypeStruct((nb, 1, BN), F32),
                 jax.ShapeDtypeStruct((nb, 1, 128), F32)],
  )(p2, hh, deg, W2, b2.reshape(1, d), Wq, Wk, Wv, if_p, gid_r)

  bn2 = 2 * BN
  nbs = n // bn2
  ms = pl.pallas_call(
      _tc4a_body,
      grid=(nbs,),
      in_specs=[
          pl.BlockSpec((1, 1, bn2), lambda i: (i, 0, 0)),
          pl.BlockSpec((1, 1, bn2), lambda i: (i, 0, 0)),
          pl.BlockSpec((nb, 1, 128), lambda i: (0, 0, 0)),
      ],
      out_specs=pl.BlockSpec((2, 128), lambda i: (0, 0)),
      out_shape=jax.ShapeDtypeStruct((2, 128), F32),
      scratch_shapes=[pltpu.VMEM((8, 128), F32)],
  )(att.reshape(nbs, 1, bn2), graph_ids.astype(I32).reshape(nbs, 1, bn2), mp)

  ro = pl.pallas_call(
      _tc4b_body,
      grid=(nb,),
      in_specs=[
          _row_spec(d),
          _row_spec(d),
          blk_spec,
          blk_spec,
          _full_spec((2, 128)),
          _full_spec((d, d)),
          _full_spec(ffW1.shape),
          _full_spec((1, ffW1.shape[1])),
          _full_spec(ffW2.shape),
          _full_spec((1, d)),
          _full_spec((1, d)),
          _full_spec((1, d)),
      ],
      out_specs=pl.BlockSpec((128, 128), lambda i: (0, 0)),
      out_shape=jax.ShapeDtypeStruct((128, 128), F32),
      scratch_shapes=[pltpu.VMEM((128, 128), F32)],
  )(hh2, v, att, gid_r, ms, Wc, ffW1, ffb1.reshape(1, -1), ffW2,
    ffb2.reshape(1, -1), ln_g.reshape(1, d), ln_b.reshape(1, d))

  return ro[:g], ia[:g]


# speedup vs baseline: 1.0216x; 1.0216x over previous
import functools

import jax
import jax.numpy as jnp
from jax import lax
from jax.experimental import pallas as pl
from jax.experimental.pallas import tpu as pltpu
from jax.experimental.pallas import tpu_sc as plsc

F32 = jnp.float32
I32 = jnp.int32

NC = 2
NS = 16
NW = NC * NS
CH = 128
CW = 16
NBUF = 2
BN = 1000


def _sc_mesh():
  return plsc.VectorSubcoreMesh(core_axis_name="c", subcore_axis_name="s")


@functools.lru_cache(None)
def _deg_call(nacc, nch):
  spl = nacc // NS

  def body(srcd, dstd, ones_h, zeros_h, out, acc_o, acc_i):
    cid = lax.axis_index("c")
    sid = lax.axis_index("s")
    wid = sid * NC + cid

    def inner(idxs, idxd, ones_v, zeros_v):
      pltpu.sync_copy(ones_h, ones_v)
      pltpu.sync_copy(zeros_h, zeros_v)
      base = sid * spl
      pltpu.sync_copy(zeros_v, acc_o.at[pl.ds(base, spl)])
      pltpu.sync_copy(zeros_v, acc_i.at[pl.ds(base, spl)])
      plsc.subcore_barrier()
      pltpu.sync_copy(srcd.at[wid], idxs)
      pltpu.sync_copy(dstd.at[wid], idxd)

      def step(j, carry):
        pltpu.sync_copy(ones_v, acc_o.at[idxs.at[j]], add=True)
        pltpu.sync_copy(ones_v, acc_i.at[idxd.at[j]], add=True)
        return carry

      lax.fori_loop(0, nch, step, 0)
      plsc.subcore_barrier()
      pltpu.sync_copy(acc_o.at[pl.ds(base, spl)],
                      out.at[pl.ds((cid * 2) * nacc + base, spl)])
      pltpu.sync_copy(acc_i.at[pl.ds(base, spl)],
                      out.at[pl.ds((cid * 2 + 1) * nacc + base, spl)])

    pl.run_scoped(inner,
                  pltpu.VMEM((nch, CH), I32),
                  pltpu.VMEM((nch, CH), I32),
                  pltpu.VMEM((CH,), F32),
                  pltpu.VMEM((spl,), F32))

  return pl.kernel(
      body,
      out_type=jax.ShapeDtypeStruct((4 * nacc,), F32),
      mesh=_sc_mesh(),
      scratch_types=[
          pltpu.VMEM_SHARED((nacc,), F32),
          pltpu.VMEM_SHARED((nacc,), F32),
      ],
  )


@functools.lru_cache(None)
def _conv_call(n, d, nacc, nch):
  spl = nacc // NS
  nz = spl // CH

  def body(xs, srcg, dstd, zeros_h, out, acc):
    cid = lax.axis_index("c")
    sid = lax.axis_index("s")
    wid = sid * NC + cid

    nwin = nch // CW

    def inner(idxs, idxd, rows, semg, sems0, sems1):
      sems = [sems0, sems1]
      rbase = sid * spl
      for k in range(nz):
        pltpu.sync_copy(zeros_h, acc.at[pl.ds(rbase + k * CH, CH)])
      plsc.subcore_barrier()

      def gstart(wp, k, b):
        pltpu.async_copy(xs.at[idxs.at[wp, k]], rows.at[b], semg)

      def gwait(b):
        pltpu.make_async_copy(xs.at[idxs.at[0, 0]], rows.at[b], semg).wait()

      def sstart(wp, k, b):
        pltpu.async_copy(rows.at[b], acc.at[idxd.at[wp, k]], sems[b], add=True)

      def swait(b):
        pltpu.make_async_copy(rows.at[b], acc.at[idxd.at[0, 0]],
                              sems[b]).wait()

      pltpu.sync_copy(srcg.at[wid, pl.ds(0, CW)], idxs.at[0])
      pltpu.sync_copy(dstd.at[wid, pl.ds(0, CW)], idxd.at[0])
      gstart(0, 0, 0)

      def win(w, carry):
        wp = w % 2

        @pl.when(w + 1 < nwin)
        def _():
          pltpu.sync_copy(srcg.at[wid, pl.ds((w + 1) * CW, CW)],
                          idxs.at[1 - wp])
          pltpu.sync_copy(dstd.at[wid, pl.ds((w + 1) * CW, CW)],
                          idxd.at[1 - wp])

        for k in range(CW):
          b = k % NBUF
          gwait(b)
          if k == 0:
            @pl.when(w > 0)
            def _():
              swait(1 - b)
          else:
            swait(1 - b)
          if k + 1 < CW:
            gstart(wp, k + 1, 1 - b)
          else:
            @pl.when(w + 1 < nwin)
            def _():
              gstart(1 - wp, 0, 1 - b)
          sstart(wp, k, b)
        return carry

      lax.fori_loop(0, nwin, win, 0)
      swait((CW - 1) % NBUF)
      plsc.subcore_barrier()
      obase = cid * nacc + rbase
      for k in range(nz):
        pltpu.sync_copy(acc.at[pl.ds(rbase + k * CH, CH)],
                        out.at[pl.ds(obase + k * CH, CH)])

    pl.run_scoped(inner,
                  pltpu.VMEM((2, CW, CH), I32),
                  pltpu.VMEM((2, CW, CH), I32),
                  pltpu.VMEM((NBUF, CH, d), F32),
                  pltpu.SemaphoreType.DMA,
                  pltpu.SemaphoreType.DMA,
                  pltpu.SemaphoreType.DMA)

  return pl.kernel(
      body,
      out_type=jax.ShapeDtypeStruct((2 * nacc, d), F32),
      mesh=_sc_mesh(),
      compiler_params=pltpu.CompilerParams(use_tc_tiling_on_sc=True),
      scratch_types=[
          pltpu.VMEM_SHARED((nacc, d), F32),
      ],
  )


_BN_SCALE = float(1.0 / (1.0 + 1e-5) ** 0.5)
_ATT_SCALE = float(1.0 / 1280.0 ** 0.5)


def _tc1_body(h_ref, deg_ref, g_ref, b_ref, gid_ref, xs_ref, ia_ref,
              iacc, cnt):
  i = pl.program_id(0)
  nb = pl.num_programs(0)
  dsl = deg_ref[0]
  inv_out = lax.rsqrt(jnp.maximum(dsl[0] + dsl[2], 1.0))
  hb = h_ref[...]
  bnh = hb * (g_ref[0] * _BN_SCALE) + b_ref[0]
  xs_ref[...] = bnh * inv_out[:, None]

  gid = gid_ref[0, 0]
  ohf = (gid[:, None] == lax.broadcasted_iota(I32, (BN, 128), 1)).astype(F32)

  @pl.when(i == 0)
  def _():
    iacc[...] = jnp.zeros((128, 128), F32)
    cnt[...] = jnp.zeros((8, 128), F32)

  iacc[...] = iacc[...] + lax.dot_general(
      ohf, hb, (((0,), (0,)), ((), ())), preferred_element_type=F32)
  cnt[0:1, :] = cnt[0:1, :] + jnp.sum(ohf, axis=0, keepdims=True)

  @pl.when(i == nb - 1)
  def _():
    ng = ia_ref.shape[0]
    ia_ref[...] = (iacc[...] / jnp.maximum(cnt[0], 1.0)[:, None])[0:ng]


def _tc2_body(h_ref, p_ref, deg_ref, w1_ref, b1_ref, g2_ref, b2_ref,
              hh_ref, xs2_ref):
  dsl = deg_ref[0]
  inv_out = lax.rsqrt(jnp.maximum(dsl[0] + dsl[2], 1.0))
  inv_in = lax.rsqrt(jnp.maximum(dsl[1] + dsl[3], 1.0))
  p = p_ref[...]
  agg = (p[0] + p[1]) * inv_in[:, None]
  z = jnp.dot(agg, w1_ref[...], preferred_element_type=F32) + b1_ref[0]
  hh = h_ref[...] + jnp.maximum(z, 0.0)
  hh_ref[...] = hh
  xs2_ref[...] = (hh * (g2_ref[0] * _BN_SCALE) + b2_ref[0]) * inv_out[:, None]


def _tc3_body(p_ref, hh_ref, deg_ref, w2_ref, b2_ref, wq_ref, wk_ref, wv_ref,
              if_ref, gid_ref, hh2_ref, v_ref, att_ref, mp_ref):
  dsl = deg_ref[0]
  inv_in = lax.rsqrt(jnp.maximum(dsl[1] + dsl[3], 1.0))
  p = p_ref[...]
  agg = (p[0] + p[1]) * inv_in[:, None]
  z = jnp.dot(agg, w2_ref[...], preferred_element_type=F32) + b2_ref[0]
  hh2 = hh_ref[...] + jnp.maximum(z, 0.0)
  q = jnp.dot(hh2, wq_ref[...], preferred_element_type=F32)
  v = jnp.dot(hh2, wv_ref[...], preferred_element_type=F32)
  kg = jnp.dot(if_ref[...], wk_ref[...], preferred_element_type=F32)
  gid = gid_ref[0, 0]
  oh = gid[:, None] == lax.broadcasted_iota(I32, (BN, 128), 1)
  ohf = oh.astype(F32)
  kn = jnp.dot(ohf, kg, preferred_element_type=F32)
  att = jnp.sum(q * kn, axis=1) * _ATT_SCALE
  att_ref[0, 0] = att
  mp_ref[0, 0] = jnp.max(jnp.where(oh, att[:, None], -1e30), axis=0)
  hh2_ref[...] = hh2
  v_ref[...] = v


def _ln(x, g, b):
  mu = jnp.mean(x, axis=-1, keepdims=True)
  dxm = x - mu
  var = jnp.mean(dxm * dxm, axis=-1, keepdims=True)
  return dxm * lax.rsqrt(var + 1e-5) * g + b


def _tc4_body(hh2_ref, v_ref, att_ref, gid_ref, mp_ref, wc_ref,
              fw1_ref, fb1_ref, fw2_ref, fb2_ref, lg_ref, lb_ref,
              ro_ref, stat, racc):
  p = pl.program_id(0)
  i = pl.program_id(1)
  nb = pl.num_programs(1)
  gid = gid_ref[0, 0]
  ohf = (gid[:, None] == lax.broadcasted_iota(I32, (BN, 128), 1)).astype(F32)

  @pl.when((p == 0) & (i == 0))
  def _():
    stat[0:1, :] = jnp.max(mp_ref[:, 0, :], axis=0, keepdims=True)
    stat[1:2, :] = jnp.zeros((1, 128), F32)
    racc[...] = jnp.zeros((128, 128), F32)

  att = att_ref[0, 0]
  mg = jnp.dot(ohf, stat[0], preferred_element_type=F32)
  e = jnp.exp(att - mg)

  @pl.when(p == 0)
  def _():
    stat[1:2, :] = stat[1:2, :] + jnp.dot(e, ohf,
                                          preferred_element_type=F32)[None, :]

  @pl.when(p == 1)
  def _():
    s_g = jnp.dot(ohf, stat[1], preferred_element_type=F32)
    alpha = e / s_g
    tp = v_ref[...] * alpha[:, None]
    mo = jnp.dot(tp, wc_ref[...], preferred_element_type=F32)
    mo = _ln(mo + hh2_ref[...], lg_ref[0], lb_ref[0])
    ffo = jnp.dot(
        jnp.maximum(
            jnp.dot(mo, fw1_ref[...], preferred_element_type=F32) + fb1_ref[0],
            0.0),
        fw2_ref[...], preferred_element_type=F32) + fb2_ref[0]
    mo2 = _ln(ffo + mo, lg_ref[0], lb_ref[0])
    racc[...] = racc[...] + lax.dot_general(
        ohf, mo2, (((0,), (0,)), ((), ())), preferred_element_type=F32)

    @pl.when(i == nb - 1)
    def _():
      ro_ref[...] = racc[0:ro_ref.shape[0]]


def _row_spec(d):
  return pl.BlockSpec((BN, d), lambda i: (i, 0))


def _full_spec(shape):
  return pl.BlockSpec(shape, lambda i: tuple(0 for _ in shape))


def kernel(h, inter_f, edge_index, graph_ids, W1, b1, W2, b2, bn1_g, bn1_b,
           bn2_g, bn2_b, Wq, Wk, Wv, Wc, ffW1, ffb1, ffW2, ffb2, ln_g, ln_b):
  n, d = h.shape
  g = inter_f.shape[0]
  e = edge_index.shape[1]
  nb = n // BN

  nacc = -(-(n + 64) // (NS * CH)) * (NS * CH)
  nch = -(-e // (NW * CH))
  nch = -(-nch // CW) * CW
  ep = NW * nch * CH
  pad = ep - e

  src = edge_index[0].astype(I32)
  dst = edge_index[1].astype(I32)
  padi = jnp.arange(pad, dtype=I32) % 32
  srcg = jnp.concatenate([src, padi]).reshape(NW, nch, CH)
  srcd = jnp.concatenate([src, n + padi]).reshape(NW, nch, CH)
  dstd = jnp.concatenate([dst, n + 32 + padi]).reshape(NW, nch, CH)

  deg = _deg_call(nacc, nch)(
      srcd, dstd, jnp.ones((CH,), F32), jnp.zeros((nacc // NS,), F32))
  deg = deg.reshape(4, nacc)[:, :n].reshape(4, nb, BN).transpose(1, 0, 2)

  deg_spec = pl.BlockSpec((1, 4, BN), lambda i: (i, 0, 0))
  vec_spec = _full_spec((1, d))
  gid_r = graph_ids.astype(I32).reshape(nb, 1, BN)
  blk_spec = pl.BlockSpec((1, 1, BN), lambda i: (i, 0, 0))

  xs1, ia = pl.pallas_call(
      _tc1_body,
      grid=(nb,),
      in_specs=[_row_spec(d), deg_spec, vec_spec, vec_spec, blk_spec],
      out_specs=[_row_spec(d), _full_spec((g, 128))],
      out_shape=[jax.ShapeDtypeStruct((n, d), F32),
                 jax.ShapeDtypeStruct((g, 128), F32)],
      scratch_shapes=[pltpu.VMEM((128, 128), F32), pltpu.VMEM((8, 128), F32)],
  )(h, deg, bn1_g.reshape(1, d), bn1_b.reshape(1, d), gid_r)

  zeros_chd = jnp.zeros((CH, d), F32)
  p1 = _conv_call(n, d, nacc, nch)(xs1, srcg, dstd, zeros_chd)
  p1 = p1.reshape(2, nacc, d)
  p_spec = pl.BlockSpec((2, BN, d), lambda i: (0, i, 0))

  hh, xs2 = pl.pallas_call(
      _tc2_body,
      grid=(nb,),
      in_specs=[_row_spec(d), p_spec, deg_spec, _full_spec((d, d)), vec_spec,
                vec_spec, vec_spec],
      out_specs=[_row_spec(d), _row_spec(d)],
      out_shape=[jax.ShapeDtypeStruct((n, d), F32),
                 jax.ShapeDtypeStruct((n, d), F32)],
  )(h, p1, deg, W1, b1.reshape(1, d), bn2_g.reshape(1, d),
    bn2_b.reshape(1, d))

  p2 = _conv_call(n, d, nacc, nch)(xs2, srcg, dstd, zeros_chd)
  p2 = p2.reshape(2, nacc, d)

  if_p = jnp.zeros((128, d), F32).at[:g].set(inter_f)

  hh2, v, att, mp = pl.pallas_call(
      _tc3_body,
      grid=(nb,),
      in_specs=[p_spec, _row_spec(d), deg_spec, _full_spec((d, d)), vec_spec,
                _full_spec((d, d)), _full_spec((d, d)), _full_spec((d, d)),
                _full_spec((128, d)), blk_spec],
      out_specs=[_row_spec(d), _row_spec(d), blk_spec,
                 pl.BlockSpec((1, 1, 128), lambda i: (i, 0, 0))],
      out_shape=[jax.ShapeDtypeStruct((n, d), F32),
                 jax.ShapeDtypeStruct((n, d), F32),
                 jax.ShapeDtypeStruct((nb, 1, BN), F32),
                 jax.ShapeDtypeStruct((nb, 1, 128), F32)],
  )(p2, hh, deg, W2, b2.reshape(1, d), Wq, Wk, Wv, if_p, gid_r)

  ro = pl.pallas_call(
      _tc4_body,
      grid=(2, nb),
      in_specs=[
          pl.BlockSpec((BN, d), lambda p, i: (p * i, 0)),
          pl.BlockSpec((BN, d), lambda p, i: (p * i, 0)),
          pl.BlockSpec((1, 1, BN), lambda p, i: (i, 0, 0)),
          pl.BlockSpec((1, 1, BN), lambda p, i: (i, 0, 0)),
          pl.BlockSpec((nb, 1, 128), lambda p, i: (0, 0, 0)),
          pl.BlockSpec((d, d), lambda p, i: (0, 0)),
          pl.BlockSpec(ffW1.shape, lambda p, i: (0, 0)),
          pl.BlockSpec((1, ffW1.shape[1]), lambda p, i: (0, 0)),
          pl.BlockSpec(ffW2.shape, lambda p, i: (0, 0)),
          pl.BlockSpec((1, d), lambda p, i: (0, 0)),
          pl.BlockSpec((1, d), lambda p, i: (0, 0)),
          pl.BlockSpec((1, d), lambda p, i: (0, 0)),
      ],
      out_specs=[pl.BlockSpec((g, 128), lambda p, i: (0, 0))],
      out_shape=[jax.ShapeDtypeStruct((g, 128), F32)],
      scratch_shapes=[pltpu.VMEM((8, 128), F32),
                      pltpu.VMEM((128, 128), F32)],
  )(hh2, v, att, gid_r, mp, Wc, ffW1, ffb1.reshape(1, -1), ffW2,
    ffb2.reshape(1, -1), ln_g.reshape(1, d), ln_b.reshape(1, d))

  return ro[0], ia
